# baseline (speedup 1.0000x reference)
import jax
import jax.numpy as jnp
from jax.experimental import pallas as pl
from jax.experimental.pallas import tpu as pltpu

IN_F = 10
TILE_B = 8192


def _mlp_kernel(xa_ref, xb_ref, w1_ref, b1_ref, w2_ref, b2_ref, o_ref):
    w1 = w1_ref[...]
    w2 = w2_ref[...]
    b1 = b1_ref[...]
    b2 = b2_ref[...]

    h = jnp.dot(xa_ref[...], w1, preferred_element_type=jnp.float32) + b1
    h = jnp.maximum(h, 0.0)
    o_ref[0] = jnp.dot(h, w2, preferred_element_type=jnp.float32) + b2

    h = jnp.dot(xb_ref[...], w1, preferred_element_type=jnp.float32) + b1
    h = jnp.maximum(h, 0.0)
    o_ref[1] = jnp.dot(h, w2, preferred_element_type=jnp.float32) + b2


def kernel(x, w1_t, b1_2d, w2_t, b2_2d):
    B = x.shape[0]
    half = B // (2 * TILE_B)
    y3 = pl.pallas_call(
        _mlp_kernel,
        out_shape=jax.ShapeDtypeStruct((2, B // 2, IN_F), x.dtype),
        grid_spec=pl.GridSpec(
            grid=(half,),
            in_specs=[
                pl.BlockSpec((TILE_B, IN_F), lambda i: (i, 0)),
                pl.BlockSpec((TILE_B, IN_F), lambda i, h=half: (i + h, 0)),
                pl.BlockSpec((IN_F, IN_F), lambda i: (0, 0)),
                pl.BlockSpec((1, IN_F), lambda i: (0, 0)),
                pl.BlockSpec((IN_F, IN_F), lambda i: (0, 0)),
                pl.BlockSpec((1, IN_F), lambda i: (0, 0)),
            ],
            out_specs=pl.BlockSpec((2, TILE_B, IN_F), lambda i: (0, i, 0)),
        ),
        compiler_params=pltpu.CompilerParams(
            dimension_semantics=("parallel",),
            vmem_limit_bytes=64 * 1024 * 1024,
        ),
        cost_estimate=pl.CostEstimate(
            flops=4 * B * IN_F * IN_F,
            transcendentals=0,
            bytes_accessed=2 * B * IN_F * 4,
        ),
    )(x, x, w1_t, b1_2d, w2_t, b2_2d)
    return jnp.reshape(y3, (B, IN_F))
